# baseline (speedup 1.0000x reference)
import functools

import jax
import jax.numpy as jnp
from jax import lax
from jax.experimental import pallas as pl
from jax.experimental.pallas import tpu as pltpu
from jax.experimental.pallas import tpu_sc as plsc

B, C, Hs, Ws = 32, 1024, 32, 32
E = 64
TOP_K = 2
HW = Hs * Ws
NROWS = B * C
NW = 32
ROWS_W = NROWS // NW
CH = 16
NBUF = 2
NCH = ROWS_W // CH

_mesh = plsc.VectorSubcoreMesh(core_axis_name="c", subcore_axis_name="s")


_SLICES = [(h, o) for h in range(Hs) for o in range(0, Ws, 16)]


@functools.partial(
    pl.kernel,
    out_type=jax.ShapeDtypeStruct((NROWS * 16,), jnp.float32),
    mesh=_mesh,
    scratch_types=[
        pltpu.VMEM((NBUF, CH, Hs, Ws), jnp.float32),
        pltpu.VMEM((ROWS_W * 16,), jnp.float32),
        pltpu.SemaphoreType.DMA((NBUF,)),
    ],
    compiler_params=pltpu.CompilerParams(use_tc_tiling_on_sc=False),
)
def _sc_rowsum(mh_hbm, x_hbm, buf, xout, sems):
    wid = lax.axis_index("s") * 2 + lax.axis_index("c")

    def chunk_src(ch):
        return mh_hbm.at[wid, pl.ds(ch * CH, CH)]

    for s in range(NBUF):
        pltpu.async_copy(chunk_src(s), buf.at[s], sems.at[s])

    @pl.loop(0, NCH, step=NBUF)
    def _chunks(ch0):
        for s in range(NBUF):
            ch = ch0 + s
            pltpu.make_async_copy(chunk_src(ch), buf.at[s], sems.at[s]).wait()
            for r in range(CH):
                accs = [
                    buf[s, r, h, pl.ds(o, 16)] for h, o in _SLICES[:8]]
                for j, (h, o) in enumerate(_SLICES[8:]):
                    accs[j % 8] = accs[j % 8] + buf[s, r, h, pl.ds(o, 16)]
                acc = ((accs[0] + accs[1]) + (accs[2] + accs[3])) + (
                    (accs[4] + accs[5]) + (accs[6] + accs[7]))
                xout[pl.ds(pl.multiple_of((ch * CH + r) * 16, 16), 16)] = acc

            nxt = ch + NBUF

            @pl.when(nxt < NCH)
            def _prefetch():
                pltpu.async_copy(chunk_src(nxt), buf.at[s], sems.at[s])

    pltpu.sync_copy(xout, x_hbm.at[pl.ds(wid * ROWS_W * 16, ROWS_W * 16)])


def _tc_epilogue(x_ref, noise_ref, wr_ref, br_ref, wn_ref, bn_ref,
                 router_ref, idx_ref, noisy_ref):
    sums = jnp.sum(x_ref[...], axis=2)
    inv_hw = jnp.float32(1.0 / HW)
    dims = (((1,), (1,)), ((), ()))
    route_logits = jax.lax.dot_general(
        sums, wr_ref[...], dims, preferred_element_type=jnp.float32,
        precision=jax.lax.Precision.HIGHEST) * inv_hw + br_ref[...]
    noise_logits = jax.lax.dot_general(
        sums, wn_ref[...], dims, preferred_element_type=jnp.float32,
        precision=jax.lax.Precision.HIGHEST) * inv_hw + bn_ref[...]

    def softmax(v):
        m = jnp.max(v, axis=1, keepdims=True)
        e = jnp.exp(v - m)
        return e / jnp.sum(e, axis=1, keepdims=True)

    logits = softmax(route_logits)
    n = softmax(noise_ref[...] * jax.nn.softplus(noise_logits))
    noisy = logits + n
    noisy_ref[...] = noisy

    iota = jax.lax.broadcasted_iota(jnp.int32, (B, E), 1)
    big = jnp.int32(E)
    v1 = jnp.max(noisy, axis=1, keepdims=True)
    i1 = jnp.min(jnp.where(noisy == v1, iota, big), axis=1, keepdims=True)
    masked = jnp.where(iota == i1, -jnp.inf, noisy)
    v2 = jnp.max(masked, axis=1, keepdims=True)
    i2 = jnp.min(jnp.where(masked == v2, iota, big), axis=1, keepdims=True)

    iota2 = jax.lax.broadcasted_iota(jnp.int32, (B, TOP_K), 1)
    idx_ref[...] = jnp.where(iota2 == 0, i1, i2)
    e2 = jnp.exp(v2 - v1)
    denom = 1.0 + e2
    router_ref[...] = jnp.where(iota2 == 0, 1.0 / denom, e2 / denom)


@jax.jit
def kernel(mh_output, noise, W_route, b_route, W_noise, b_noise):
    x = _sc_rowsum(mh_output).reshape(B, C, 16)
    br = b_route.reshape(1, E)
    bn = b_noise.reshape(1, E)
    router_output, indices, noisy_logits = pl.pallas_call(
        _tc_epilogue,
        out_shape=[
            jax.ShapeDtypeStruct((B, TOP_K), jnp.float32),
            jax.ShapeDtypeStruct((B, TOP_K), jnp.int32),
            jax.ShapeDtypeStruct((B, E), jnp.float32),
        ],
    )(x, noise, W_route, br, W_noise, bn)
    return (router_output, indices, noisy_logits)


# speedup vs baseline: 1.0764x; 1.0764x over previous
import jax
import jax.numpy as jnp
from jax.experimental import pallas as pl
from jax.experimental.pallas import tpu as pltpu

B, C, Hs, Ws = 32, 1024, 32, 32
E = 64
TOP_K = 2
HW = Hs * Ws
C_BLK = 16
NC = C // C_BLK


def _router_kernel(mh_ref, noise_ref, wrt_ref, br_ref, wnt_ref, bn_ref,
                   router_ref, idx_ref, noisy_ref, acc_r, acc_n):
    c = pl.program_id(0)

    @pl.when(c == 0)
    def _init():
        acc_r[...] = jnp.zeros_like(acc_r)
        acc_n[...] = jnp.zeros_like(acc_n)

    x_part = jnp.sum(mh_ref[...], axis=(2, 3))
    dims = (((1,), (0,)), ((), ()))
    acc_r[...] += jax.lax.dot_general(
        x_part, wrt_ref[...], dims, preferred_element_type=jnp.float32,
        precision=jax.lax.Precision.HIGHEST)
    acc_n[...] += jax.lax.dot_general(
        x_part, wnt_ref[...], dims, preferred_element_type=jnp.float32,
        precision=jax.lax.Precision.HIGHEST)

    @pl.when(c == NC - 1)
    def _epilogue():
        inv_hw = jnp.float32(1.0 / HW)
        route_logits = acc_r[...] * inv_hw + br_ref[...]
        noise_logits = acc_n[...] * inv_hw + bn_ref[...]

        def softmax(v):
            m = jnp.max(v, axis=1, keepdims=True)
            e = jnp.exp(v - m)
            return e / jnp.sum(e, axis=1, keepdims=True)

        logits = softmax(route_logits)
        n = softmax(noise_ref[...] * jax.nn.softplus(noise_logits))
        noisy = logits + n
        noisy_ref[...] = noisy

        iota = jax.lax.broadcasted_iota(jnp.int32, (B, E), 1)
        big = jnp.int32(E)
        v1 = jnp.max(noisy, axis=1, keepdims=True)
        i1 = jnp.min(jnp.where(noisy == v1, iota, big), axis=1, keepdims=True)
        masked = jnp.where(iota == i1, -jnp.inf, noisy)
        v2 = jnp.max(masked, axis=1, keepdims=True)
        i2 = jnp.min(jnp.where(masked == v2, iota, big), axis=1, keepdims=True)

        iota2 = jax.lax.broadcasted_iota(jnp.int32, (B, TOP_K), 1)
        idx_ref[...] = jnp.where(iota2 == 0, i1, i2)
        e2 = jnp.exp(v2 - v1)
        denom = 1.0 + e2
        router_ref[...] = jnp.where(iota2 == 0, 1.0 / denom, e2 / denom)


@jax.jit
def kernel(mh_output, noise, W_route, b_route, W_noise, b_noise):
    br = b_route.reshape(1, E)
    bn = b_noise.reshape(1, E)
    wrt = W_route.T
    wnt = W_noise.T

    router_output, indices, noisy_logits = pl.pallas_call(
        _router_kernel,
        grid=(NC,),
        in_specs=[
            pl.BlockSpec((B, C_BLK, Hs, Ws), lambda c: (0, c, 0, 0)),
            pl.BlockSpec((B, E), lambda c: (0, 0)),
            pl.BlockSpec((C_BLK, E), lambda c: (c, 0)),
            pl.BlockSpec((1, E), lambda c: (0, 0)),
            pl.BlockSpec((C_BLK, E), lambda c: (c, 0)),
            pl.BlockSpec((1, E), lambda c: (0, 0)),
        ],
        out_specs=[
            pl.BlockSpec((B, TOP_K), lambda c: (0, 0)),
            pl.BlockSpec((B, TOP_K), lambda c: (0, 0)),
            pl.BlockSpec((B, E), lambda c: (0, 0)),
        ],
        out_shape=[
            jax.ShapeDtypeStruct((B, TOP_K), jnp.float32),
            jax.ShapeDtypeStruct((B, TOP_K), jnp.int32),
            jax.ShapeDtypeStruct((B, E), jnp.float32),
        ],
        scratch_shapes=[
            pltpu.VMEM((B, E), jnp.float32),
            pltpu.VMEM((B, E), jnp.float32),
        ],
    )(mh_output, noise, wrt, br, wnt, bn)
    return (router_output, indices, noisy_logits)


# speedup vs baseline: 3.7951x; 3.5258x over previous
import functools

import jax
import jax.numpy as jnp
from jax.experimental import pallas as pl
from jax.experimental.pallas import tpu as pltpu

B, C, Hs, Ws = 32, 1024, 32, 32
E = 64
TOP_K = 2
HW = Hs * Ws
C_BLK = 128
NC = C // C_BLK


NSPLIT = 4
B_SPL = B // NSPLIT


def _router_kernel(mh0_ref, mh1_ref, mh2_ref, mh3_ref,
                   noise_ref, wr_ref, br_ref, wn_ref, bn_ref,
                   router_ref, idx_ref, noisy_ref, acc_r, acc_n):
    c = pl.program_id(0)

    @pl.when(c == 0)
    def _init():
        acc_r[...] = jnp.zeros_like(acc_r)
        acc_n[...] = jnp.zeros_like(acc_n)

    x_part = jnp.concatenate(
        [jnp.sum(r[...], axis=2) for r in (mh0_ref, mh1_ref, mh2_ref, mh3_ref)],
        axis=0)
    dims = (((1,), (1,)), ((), ()))
    acc_r[...] += jax.lax.dot_general(
        x_part, wr_ref[...], dims, preferred_element_type=jnp.float32,
        precision=jax.lax.Precision.HIGHEST)
    acc_n[...] += jax.lax.dot_general(
        x_part, wn_ref[...], dims, preferred_element_type=jnp.float32,
        precision=jax.lax.Precision.HIGHEST)

    @pl.when(c == NC - 1)
    def _epilogue():
        inv_hw = jnp.float32(1.0 / HW)
        route_logits = acc_r[...] * inv_hw + br_ref[...]
        noise_logits = acc_n[...] * inv_hw + bn_ref[...]

        def softmax(v):
            m = jnp.max(v, axis=1, keepdims=True)
            e = jnp.exp(v - m)
            return e / jnp.sum(e, axis=1, keepdims=True)

        logits = softmax(route_logits)
        n = softmax(noise_ref[...] * jax.nn.softplus(noise_logits))
        noisy = logits + n
        noisy_ref[...] = noisy

        iota = jax.lax.broadcasted_iota(jnp.int32, (B, E), 1)
        big = jnp.int32(E)
        v1 = jnp.max(noisy, axis=1, keepdims=True)
        i1 = jnp.min(jnp.where(noisy == v1, iota, big), axis=1, keepdims=True)
        masked = jnp.where(iota == i1, -jnp.inf, noisy)
        v2 = jnp.max(masked, axis=1, keepdims=True)
        i2 = jnp.min(jnp.where(masked == v2, iota, big), axis=1, keepdims=True)

        iota2 = jax.lax.broadcasted_iota(jnp.int32, (B, TOP_K), 1)
        idx_ref[...] = jnp.where(iota2 == 0, i1, i2)
        e2 = jnp.exp(v2 - v1)
        denom = 1.0 + e2
        router_ref[...] = jnp.where(iota2 == 0, 1.0 / denom, e2 / denom)


@jax.jit
def kernel(mh_output, noise, W_route, b_route, W_noise, b_noise):
    mh = mh_output.reshape(B, C, HW)
    br = b_route.reshape(1, E)
    bn = b_noise.reshape(1, E)

    grid = (NC,)
    router_output, indices, noisy_logits = pl.pallas_call(
        _router_kernel,
        grid=grid,
        in_specs=[
            pl.BlockSpec((B_SPL, C_BLK, HW),
                         functools.partial(lambda i, c: (i, c, 0), 0)),
            pl.BlockSpec((B_SPL, C_BLK, HW),
                         functools.partial(lambda i, c: (i, c, 0), 1)),
            pl.BlockSpec((B_SPL, C_BLK, HW),
                         functools.partial(lambda i, c: (i, c, 0), 2)),
            pl.BlockSpec((B_SPL, C_BLK, HW),
                         functools.partial(lambda i, c: (i, c, 0), 3)),
            pl.BlockSpec((B, E), lambda c: (0, 0)),
            pl.BlockSpec((E, C_BLK), lambda c: (0, c)),
            pl.BlockSpec((1, E), lambda c: (0, 0)),
            pl.BlockSpec((E, C_BLK), lambda c: (0, c)),
            pl.BlockSpec((1, E), lambda c: (0, 0)),
        ],
        out_specs=[
            pl.BlockSpec((B, TOP_K), lambda c: (0, 0)),
            pl.BlockSpec((B, TOP_K), lambda c: (0, 0)),
            pl.BlockSpec((B, E), lambda c: (0, 0)),
        ],
        out_shape=[
            jax.ShapeDtypeStruct((B, TOP_K), jnp.float32),
            jax.ShapeDtypeStruct((B, TOP_K), jnp.int32),
            jax.ShapeDtypeStruct((B, E), jnp.float32),
        ],
        scratch_shapes=[
            pltpu.VMEM((B, E), jnp.float32),
            pltpu.VMEM((B, E), jnp.float32),
        ],
    )(mh, mh, mh, mh, noise, W_route, br, W_noise, bn)
    return (router_output, indices, noisy_logits)
